# baseline (speedup 1.0000x reference)
import jax
import jax.numpy as jnp
from jax import lax
from jax.experimental import pallas as pl
from jax.experimental.pallas import tpu as pltpu
from jax.experimental.pallas import tpu_sc as plsc

BATCH = 16384
N_FIELDS = 26
LANES = 16
NUM_CORES = 2
NUM_SUBCORES = 16
NUM_WORKERS = NUM_CORES * NUM_SUBCORES
ROWS_W = BATCH // NUM_WORKERS
PER_W = ROWS_W * N_FIELDS
CHUNK = 128
N_CHUNKS = PER_W // CHUNK


def _fm_linear_body(x_hbm, table_hbm, bias_hbm, out_hbm,
                    idx_v, vals_v, out_v, bias_v, sem):
    wid = lax.axis_index("s") * NUM_CORES + lax.axis_index("c")
    base = wid * PER_W

    pltpu.sync_copy(x_hbm.at[pl.ds(base, PER_W)], idx_v)
    pltpu.sync_copy(bias_hbm, bias_v)

    def _start(j, carry):
        off = j * CHUNK
        pltpu.make_async_copy(
            table_hbm.at[idx_v.at[pl.ds(off, CHUNK)]],
            vals_v.at[pl.ds(off, CHUNK)],
            sem,
        ).start()
        return carry

    lax.fori_loop(0, N_CHUNKS, _start, 0)

    pltpu.make_async_copy(out_hbm.at[pl.ds(0, PER_W)], vals_v, sem).wait()

    bias_vec = bias_v[...]
    lane = lax.iota(jnp.int32, LANES)

    def _reduce(c, carry):
        p = (c * LANES + lane) * N_FIELDS
        acc = bias_vec
        for f in range(N_FIELDS):
            acc = acc + plsc.load_gather(vals_v, [p + f])
        out_v[pl.ds(c * LANES, LANES)] = acc
        return carry

    lax.fori_loop(0, ROWS_W // LANES, _reduce, 0)

    pltpu.sync_copy(out_v, out_hbm.at[pl.ds(wid * ROWS_W, ROWS_W)])


@jax.jit
def _fm_linear(x_flat, table_flat, bias16):
    mesh = plsc.VectorSubcoreMesh(core_axis_name="c", subcore_axis_name="s")
    call = pl.kernel(
        _fm_linear_body,
        out_type=jax.ShapeDtypeStruct((BATCH,), jnp.float32),
        mesh=mesh,
        compiler_params=pltpu.CompilerParams(needs_layout_passes=False),
        scratch_types=[
            pltpu.VMEM((PER_W,), jnp.int32),
            pltpu.VMEM((PER_W,), jnp.float32),
            pltpu.VMEM((ROWS_W,), jnp.float32),
            pltpu.VMEM((LANES,), jnp.float32),
            pltpu.SemaphoreType.DMA,
        ],
    )
    return call(x_flat, table_flat, bias16)


def kernel(x, linear_weight, bias):
    x_flat = x.astype(jnp.int32).reshape(-1)
    table_flat = linear_weight.reshape(-1)
    bias16 = jnp.broadcast_to(bias.astype(jnp.float32), (LANES,))
    out = _fm_linear(x_flat, table_flat, bias16)
    return out.reshape(BATCH, 1)


# speedup vs baseline: 1.1805x; 1.1805x over previous
import jax
import jax.numpy as jnp
from jax import lax
from jax.experimental import pallas as pl
from jax.experimental.pallas import tpu as pltpu
from jax.experimental.pallas import tpu_sc as plsc

BATCH = 16384
N_FIELDS = 26
LANES = 16
NUM_CORES = 2
NUM_SUBCORES = 16
NUM_WORKERS = NUM_CORES * NUM_SUBCORES
ROWS_W = BATCH // NUM_WORKERS
PER_W = ROWS_W * N_FIELDS
CHUNK = 128
CH_PER_F = ROWS_W // CHUNK
N_CHUNKS = N_FIELDS * CH_PER_F


def _fm_linear_body(xt_hbm, table_hbm, bias_hbm, out_hbm,
                    idx_v, vals_v, out_v, bias_v, sem):
    wid = lax.axis_index("s") * NUM_CORES + lax.axis_index("c")

    pltpu.sync_copy(xt_hbm.at[:, pl.ds(wid * ROWS_W, ROWS_W)], idx_v)
    pltpu.sync_copy(bias_hbm, bias_v)

    def _start(j, carry):
        f = j // CH_PER_F
        c = j - f * CH_PER_F
        pltpu.make_async_copy(
            table_hbm.at[idx_v.at[f, pl.ds(c * CHUNK, CHUNK)]],
            vals_v.at[pl.ds(f * ROWS_W + c * CHUNK, CHUNK)],
            sem,
        ).start()
        return carry

    lax.fori_loop(0, N_CHUNKS, _start, 0)

    pltpu.make_async_copy(out_hbm.at[pl.ds(0, PER_W)], vals_v, sem).wait()

    bias_vec = bias_v[...]

    def _reduce(c, carry):
        acc = bias_vec
        for f in range(N_FIELDS):
            acc = acc + vals_v[pl.ds(f * ROWS_W + c * LANES, LANES)]
        out_v[pl.ds(c * LANES, LANES)] = acc
        return carry

    lax.fori_loop(0, ROWS_W // LANES, _reduce, 0)

    pltpu.sync_copy(out_v, out_hbm.at[pl.ds(wid * ROWS_W, ROWS_W)])


@jax.jit
def _fm_linear(xt, table_flat, bias16):
    mesh = plsc.VectorSubcoreMesh(core_axis_name="c", subcore_axis_name="s")
    call = pl.kernel(
        _fm_linear_body,
        out_type=jax.ShapeDtypeStruct((BATCH,), jnp.float32),
        mesh=mesh,
        scratch_types=[
            pltpu.VMEM((N_FIELDS, ROWS_W), jnp.int32),
            pltpu.VMEM((PER_W,), jnp.float32),
            pltpu.VMEM((ROWS_W,), jnp.float32),
            pltpu.VMEM((LANES,), jnp.float32),
            pltpu.SemaphoreType.DMA,
        ],
    )
    return call(xt, table_flat, bias16)


def kernel(x, linear_weight, bias):
    xt = x.astype(jnp.int32).T
    table_flat = jnp.pad(linear_weight, ((0, 52), (0, 0))).reshape(-1)
    bias16 = jnp.broadcast_to(bias.astype(jnp.float32), (LANES,))
    out = _fm_linear(xt, table_flat, bias16)
    return out.reshape(BATCH, 1)


# speedup vs baseline: 1.1816x; 1.0010x over previous
import jax
import jax.numpy as jnp
from jax import lax
from jax.experimental import pallas as pl
from jax.experimental.pallas import tpu as pltpu
from jax.experimental.pallas import tpu_sc as plsc

BATCH = 16384
N_FIELDS = 26
LANES = 16
NUM_CORES = 2
NUM_SUBCORES = 16
NUM_WORKERS = NUM_CORES * NUM_SUBCORES
ROWS_W = BATCH // NUM_WORKERS
PER_W = ROWS_W * N_FIELDS
CHUNK = 128
CH_PER_F = ROWS_W // CHUNK
N_CHUNKS = N_FIELDS * CH_PER_F


def _fm_linear_body(xt_hbm, table_hbm, bias_hbm, out_hbm,
                    idx_v, vals_v, out_v, bias_v, sem):
    wid = lax.axis_index("s") * NUM_CORES + lax.axis_index("c")

    pltpu.sync_copy(xt_hbm.at[:, pl.ds(wid * ROWS_W, ROWS_W)], idx_v)
    pltpu.sync_copy(bias_hbm, bias_v)

    def _start(j, carry):
        f = j // CH_PER_F
        c = j - f * CH_PER_F
        pltpu.make_async_copy(
            table_hbm.at[idx_v.at[f, pl.ds(c * CHUNK, CHUNK)]],
            vals_v.at[pl.ds(f * ROWS_W + c * CHUNK, CHUNK)],
            sem,
        ).start()
        return carry

    lax.fori_loop(0, N_CHUNKS, _start, 0)

    pltpu.make_async_copy(out_hbm.at[pl.ds(0, PER_W)], vals_v, sem).wait()

    bias_vec = bias_v[...]

    def _reduce(c, carry):
        acc = bias_vec
        for f in range(N_FIELDS):
            acc = acc + vals_v[pl.ds(f * ROWS_W + c * LANES, LANES)]
        out_v[pl.ds(c * LANES, LANES)] = acc
        return carry

    lax.fori_loop(0, ROWS_W // LANES, _reduce, 0)

    pltpu.sync_copy(out_v, out_hbm.at[pl.ds(wid * ROWS_W, ROWS_W)])


@jax.jit
def _fm_linear(xt, table_flat, bias16):
    mesh = plsc.VectorSubcoreMesh(core_axis_name="c", subcore_axis_name="s")
    call = pl.kernel(
        _fm_linear_body,
        out_type=jax.ShapeDtypeStruct((BATCH,), jnp.float32),
        mesh=mesh,
        scratch_types=[
            pltpu.VMEM((N_FIELDS, ROWS_W), jnp.int32),
            pltpu.VMEM((PER_W,), jnp.float32),
            pltpu.VMEM((ROWS_W,), jnp.float32),
            pltpu.VMEM((LANES,), jnp.float32),
            pltpu.SemaphoreType.DMA,
        ],
    )
    return call(xt, table_flat, bias16)


def kernel(x, linear_weight, bias):
    xt = x.astype(jnp.int32).T
    table_flat = lax.optimization_barrier(linear_weight.T).reshape(-1)
    bias16 = jnp.broadcast_to(bias.astype(jnp.float32), (LANES,))
    out = _fm_linear(xt, table_flat, bias16)
    return out.reshape(BATCH, 1)
